# baseline (speedup 1.0000x reference)
import functools

import jax
import jax.numpy as jnp
from jax.experimental import pallas as pl
from jax.experimental.pallas import tpu as pltpu
from jax.experimental.pallas import tpu_sc as plsc

B = 16384
D = 256
K = 8192

BM = 1024
BK = 1024
NB = B // BM
NK = K // BK

SC_CORES = 2
SC_SUBCORES = 16
NW = SC_CORES * SC_SUBCORES
BPW = B // NW
CHUNK = 256
NCHUNK = BPW // CHUNK


def _vq_tc_body(x_ref, cb_ref, x2_ref, c2_ref, idx_ref, loss_ref,
                minv_ref, arg_ref):
    b = pl.program_id(0)
    k = pl.program_id(1)

    x = x_ref[...]
    cb = cb_ref[...]
    mm = jax.lax.dot_general(
        x, cb, (((1,), (1,)), ((), ())),
        preferred_element_type=jnp.float32)
    d2 = jnp.maximum((x2_ref[...] + c2_ref[...]) - 2.0 * mm, 0.0)
    s = jnp.sqrt(d2)

    bmin = jnp.min(s, axis=1, keepdims=True)
    lidx = jax.lax.broadcasted_iota(jnp.int32, (BM, BK), 1) + k * BK
    barg = jnp.min(jnp.where(s == bmin, lidx, jnp.int32(2 ** 30)),
                   axis=1, keepdims=True)

    @pl.when(k == 0)
    def _():
        minv_ref[...] = bmin
        arg_ref[...] = barg

    @pl.when(k > 0)
    def _():
        pv = minv_ref[...]
        pa = arg_ref[...]
        upd = bmin < pv
        minv_ref[...] = jnp.where(upd, bmin, pv)
        arg_ref[...] = jnp.where(upd, barg, pa)

    @pl.when(k == NK - 1)
    def _():
        idx_ref[...] = arg_ref[...]
        mv = minv_ref[...]
        block_loss = jnp.sum(mv * mv)

        @pl.when(b == 0)
        def _():
            loss_ref[...] = jnp.full((1, 1), block_loss, jnp.float32)

        @pl.when(b > 0)
        def _():
            loss_ref[...] = loss_ref[...] + block_loss

        @pl.when(b == NB - 1)
        def _():
            loss_ref[...] = loss_ref[...] * (1.0 / (B * D))


_vq_tc = pl.pallas_call(
    _vq_tc_body,
    grid=(NB, NK),
    in_specs=[
        pl.BlockSpec((BM, D), lambda b, k: (b, 0)),
        pl.BlockSpec((BK, D), lambda b, k: (k, 0)),
        pl.BlockSpec((BM, 1), lambda b, k: (b, 0)),
        pl.BlockSpec((1, BK), lambda b, k: (0, k)),
    ],
    out_specs=[
        pl.BlockSpec((BM, 1), lambda b, k: (b, 0)),
        pl.BlockSpec((1, 1), lambda b, k: (0, 0)),
    ],
    out_shape=[
        jax.ShapeDtypeStruct((B, 1), jnp.int32),
        jax.ShapeDtypeStruct((1, 1), jnp.float32),
    ],
    scratch_shapes=[
        pltpu.VMEM((BM, 1), jnp.float32),
        pltpu.VMEM((BM, 1), jnp.int32),
    ],
)


@functools.lru_cache(maxsize=1)
def _make_sc_gather():
    @functools.partial(
        pl.kernel,
        mesh=plsc.VectorSubcoreMesh(core_axis_name="c", subcore_axis_name="s"),
        out_type=jax.ShapeDtypeStruct((B, D), jnp.float32),
        scratch_types=[
            pltpu.VMEM((CHUNK,), jnp.int32),
            pltpu.VMEM((CHUNK, D), jnp.float32),
            pltpu.SemaphoreType.DMA,
        ],
    )
    def _sc_gather(cb_hbm, idx_hbm, out_hbm, idx_v, rows_v, sem):
        wid = jax.lax.axis_index("s") * SC_CORES + jax.lax.axis_index("c")
        base = wid * BPW
        for j in range(NCHUNK):
            off = base + j * CHUNK
            pltpu.sync_copy(idx_hbm.at[pl.ds(off, CHUNK)], idx_v)
            pltpu.async_copy(cb_hbm.at[idx_v], rows_v, sem).wait()
            pltpu.sync_copy(rows_v, out_hbm.at[pl.ds(off, CHUNK)])

    return _sc_gather


def kernel(x, codebook):
    x2 = jnp.sum(x * x, axis=1, keepdims=True)
    c2 = jnp.sum(codebook * codebook, axis=1)[None, :]
    idx2d, loss = _vq_tc(x, codebook, x2, c2)
    indices = idx2d.reshape(B)
    quantized = _make_sc_gather()(codebook, indices)
    return quantized, indices, loss.reshape(())


# speedup vs baseline: 1.1142x; 1.1142x over previous
import functools

import jax
import jax.numpy as jnp
from jax.experimental import pallas as pl
from jax.experimental.pallas import tpu as pltpu
from jax.experimental.pallas import tpu_sc as plsc

B = 16384
D = 256
K = 8192

BM = 1024
BK = 1024
NB = B // BM
NK = K // BK

SC_CORES = 2
SC_SUBCORES = 16
NW = SC_CORES * SC_SUBCORES
BPW = B // NW
CHUNK = 256
NCHUNK = BPW // CHUNK


def _vq_tc_body(x_ref, cb2_ref, x2_ref, c2_ref, iota_ref, idx_ref, loss_ref,
                minv_ref, arg_ref):
    b = pl.program_id(0)
    k = pl.program_id(1)

    x = x_ref[...]
    cb2 = cb2_ref[...]
    mm2 = jax.lax.dot_general(
        x, cb2, (((1,), (1,)), ((), ())),
        preferred_element_type=jnp.float32)
    d2 = jnp.maximum((x2_ref[...] + c2_ref[...]) - mm2, 0.0)
    s = jnp.sqrt(d2)

    bmin = jnp.min(s, axis=1, keepdims=True)
    bargf = jnp.min(jnp.where(s == bmin, iota_ref[...], jnp.float32(3e8)),
                    axis=1, keepdims=True)
    barg = bargf.astype(jnp.int32) + k * BK

    @pl.when(k == 0)
    def _():
        minv_ref[...] = bmin
        arg_ref[...] = barg

    @pl.when(k > 0)
    def _():
        pv = minv_ref[...]
        pa = arg_ref[...]
        upd = bmin < pv
        minv_ref[...] = jnp.where(upd, bmin, pv)
        arg_ref[...] = jnp.where(upd, barg, pa)

    @pl.when(k == NK - 1)
    def _():
        idx_ref[...] = arg_ref[...]
        mv = minv_ref[...]
        block_loss = jnp.sum(mv * mv)

        @pl.when(b == 0)
        def _():
            loss_ref[...] = jnp.full((1, 1), block_loss, jnp.float32)

        @pl.when(b > 0)
        def _():
            loss_ref[...] = loss_ref[...] + block_loss

        @pl.when(b == NB - 1)
        def _():
            loss_ref[...] = loss_ref[...] * (1.0 / (B * D))


_vq_tc = pl.pallas_call(
    _vq_tc_body,
    grid=(NB, NK),
    in_specs=[
        pl.BlockSpec((BM, D), lambda b, k: (b, 0)),
        pl.BlockSpec((BK, D), lambda b, k: (k, 0)),
        pl.BlockSpec((BM, 1), lambda b, k: (b, 0)),
        pl.BlockSpec((1, BK), lambda b, k: (0, k)),
        pl.BlockSpec((1, BK), lambda b, k: (0, 0)),
    ],
    out_specs=[
        pl.BlockSpec((BM, 1), lambda b, k: (b, 0)),
        pl.BlockSpec((1, 1), lambda b, k: (0, 0)),
    ],
    out_shape=[
        jax.ShapeDtypeStruct((B, 1), jnp.int32),
        jax.ShapeDtypeStruct((1, 1), jnp.float32),
    ],
    scratch_shapes=[
        pltpu.VMEM((BM, 1), jnp.float32),
        pltpu.VMEM((BM, 1), jnp.int32),
    ],
)


@functools.lru_cache(maxsize=1)
def _make_sc_gather():
    @functools.partial(
        pl.kernel,
        mesh=plsc.VectorSubcoreMesh(core_axis_name="c", subcore_axis_name="s"),
        out_type=jax.ShapeDtypeStruct((B, D), jnp.float32),
        scratch_types=[
            pltpu.VMEM((CHUNK,), jnp.int32),
            pltpu.VMEM((CHUNK, D), jnp.float32),
            pltpu.SemaphoreType.DMA,
        ],
    )
    def _sc_gather(cb_hbm, idx_hbm, out_hbm, idx_v, rows_v, sem):
        wid = jax.lax.axis_index("s") * SC_CORES + jax.lax.axis_index("c")
        base = wid * BPW
        for j in range(NCHUNK):
            off = base + j * CHUNK
            pltpu.sync_copy(idx_hbm.at[pl.ds(off, CHUNK)], idx_v)
            pltpu.async_copy(cb_hbm.at[idx_v], rows_v, sem).wait()
            pltpu.sync_copy(rows_v, out_hbm.at[pl.ds(off, CHUNK)])

    return _sc_gather


def kernel(x, codebook):
    x2 = jnp.sum(x * x, axis=1, keepdims=True)
    c2 = jnp.sum(codebook * codebook, axis=1)[None, :]
    cb2 = codebook * 2.0
    iota = jnp.arange(BK, dtype=jnp.float32)[None, :]
    idx2d, loss = _vq_tc(x, cb2, x2, c2, iota)
    indices = idx2d.reshape(B)
    quantized = _make_sc_gather()(codebook, indices)
    return quantized, indices, loss.reshape(())
